# baseline (speedup 1.0000x reference)
import functools

import jax
import jax.numpy as jnp
from jax import lax
from jax.experimental import pallas as pl
from jax.experimental.pallas import tpu as pltpu
from jax.experimental.pallas import tpu_sc as plsc

_BATCH = 4096
_HIST = 200
_EMBED = 64
_B = _BATCH * _HIST
_NW = 32
_B_PER_W = _B // _NW
_HC = 2
_CROWS = _HC * 128
_NCHUNK = _HIST // _HC
_NPAIR = _NCHUNK // 2


def _make_gather():
    mesh = plsc.VectorSubcoreMesh(core_axis_name="c", subcore_axis_name="s")

    @functools.partial(
        pl.kernel,
        mesh=mesh,
        out_type=jax.ShapeDtypeStruct(
            (_HIST, _EMBED // 8, _BATCH // 128, 8, 128), jnp.float32),
        scratch_types=[
            pltpu.VMEM((_B_PER_W,), jnp.int32),
            pltpu.VMEM((2, _CROWS), jnp.int32),
            pltpu.VMEM((2, _CROWS, _EMBED), jnp.float32),
            pltpu.VMEM((2, _HC, 8, 8, 128), jnp.float32),
            pltpu.SemaphoreType.DMA,
            pltpu.SemaphoreType.DMA,
            pltpu.SemaphoreType.DMA,
            pltpu.SemaphoreType.DMA,
        ],
        compiler_params=pltpu.CompilerParams(use_tc_tiling_on_sc=False,
                                             needs_layout_passes=False),
    )
    def gather_kernel(idx_hbm, table_hbm, out_hbm, idx_v, li_v, rows_v,
                      stg_v, gsem0, gsem1, osem0, osem1):
        wid = lax.axis_index("s") * 2 + lax.axis_index("c")
        base = wid * _B_PER_W
        pltpu.sync_copy(idx_hbm.at[pl.ds(base, _B_PER_W)], idx_v)

        lane = lax.broadcasted_iota(jnp.int32, (16,), 0)
        lane200 = lane * 200
        hi8 = lane >> 3
        lo8 = lane & 7
        d1 = [hi8 + 2 * eg for eg in range(4)]

        def build_li(c, lslot):
            for h_i in range(_HC):
                for bg in range(8):
                    addr = lane200 + (bg * 16 * 200 + 2 * c + h_i)
                    v = plsc.load_gather(idx_v, [addr])
                    li_v[lslot, pl.ds(h_i * 128 + bg * 16, 16)] = v

        def gather_start(lslot, sem):
            pltpu.async_copy(table_hbm.at[li_v.at[lslot]],
                             rows_v.at[lslot], sem)

        def gather_wait(lslot, sem):
            pltpu.make_async_copy(table_hbm.at[li_v.at[lslot]],
                                  rows_v.at[lslot], sem).wait()

        def transpose(lslot):
            for h_i in range(_HC):
                dst = stg_v.at[lslot, h_i]
                for r in range(128):
                    d3 = lane * 0 + r
                    for eg in range(4):
                        v = rows_v[lslot, h_i * 128 + r, pl.ds(eg * 16, 16)]
                        plsc.store_scatter(dst, [d1[eg], lo8, d3], v)

        def out_start(c, lslot, sem):
            for h_i in range(_HC):
                pltpu.async_copy(stg_v.at[lslot, h_i],
                                 out_hbm.at[2 * c + h_i, :, wid], sem)

        def out_wait(c, lslot, sem):
            for h_i in range(_HC):
                pltpu.make_async_copy(stg_v.at[lslot, h_i],
                                      out_hbm.at[2 * c + h_i, :, wid],
                                      sem).wait()

        build_li(0, 0)
        gather_start(0, gsem0)

        def body(i, carry):
            g = 2 * i
            gather_wait(0, gsem0)
            build_li(g + 1, 1)
            gather_start(1, gsem1)

            @pl.when(i > 0)
            def _():
                out_wait(g - 2, 0, osem0)

            transpose(0)
            out_start(g, 0, osem0)

            gather_wait(1, gsem1)

            @pl.when(i + 1 < _NPAIR)
            def _():
                build_li(g + 2, 0)
                gather_start(0, gsem0)

            @pl.when(i > 0)
            def _():
                out_wait(g - 1, 1, osem1)

            transpose(1)
            out_start(g + 1, 1, osem1)
            return carry

        lax.fori_loop(0, _NPAIR, body, 0)
        out_wait(_NCHUNK - 2, 0, osem0)
        out_wait(_NCHUNK - 1, 1, osem1)

    return gather_kernel


_gather = _make_gather()


def kernel(visit_order, visit_rel_times, pos_table, time_table):
    idx = visit_rel_times.reshape(_B).astype(jnp.int32)
    out5 = _gather(idx, time_table)
    return out5.transpose(2, 4, 0, 1, 3).reshape(_BATCH, _HIST, _EMBED)


# speedup vs baseline: 1.3959x; 1.3959x over previous
import functools

import jax
import jax.numpy as jnp
from jax import lax
from jax.experimental import pallas as pl
from jax.experimental.pallas import tpu as pltpu
from jax.experimental.pallas import tpu_sc as plsc

_BATCH = 4096
_HIST = 200
_EMBED = 64
_B = _BATCH * _HIST
_NW = 32
_B_PER_W = _B // _NW
_HC = 2
_CROWS = _HC * 128
_NCHUNK = _HIST // _HC
_NPAIR = _NCHUNK // 2


def _make_gather():
    mesh = plsc.VectorSubcoreMesh(core_axis_name="c", subcore_axis_name="s")

    @functools.partial(
        pl.kernel,
        mesh=mesh,
        out_type=jax.ShapeDtypeStruct(
            (_HIST, _EMBED // 8, _BATCH // 128, 8, 128), jnp.float32),
        scratch_types=[
            pltpu.VMEM((_B_PER_W,), jnp.int32),
            pltpu.VMEM((2, _CROWS), jnp.int32),
            pltpu.VMEM((2, _CROWS, _EMBED), jnp.float32),
            pltpu.VMEM((2, _HC, 8, 8, 128), jnp.float32),
            pltpu.SemaphoreType.DMA,
            pltpu.SemaphoreType.DMA,
            pltpu.SemaphoreType.DMA,
            pltpu.SemaphoreType.DMA,
        ],
        compiler_params=pltpu.CompilerParams(use_tc_tiling_on_sc=False,
                                             needs_layout_passes=False),
    )
    def gather_kernel(idx_hbm, table_hbm, out_hbm, idx_v, li_v, rows_v,
                      stg_v, gsem0, gsem1, osem0, osem1):
        wid = lax.axis_index("s") * 2 + lax.axis_index("c")
        base = wid * _B_PER_W
        pltpu.sync_copy(idx_hbm.at[pl.ds(base, _B_PER_W)], idx_v)

        lane = lax.broadcasted_iota(jnp.int32, (16,), 0)
        lane200 = lane * 200
        hi8 = lane >> 3
        lo8 = lane & 7
        d1 = [hi8 + 2 * eg for eg in range(4)]

        def build_li(c, lslot):
            for h_i in range(_HC):
                for bg in range(8):
                    addr = lane200 + (bg * 16 * 200 + 2 * c + h_i)
                    v = plsc.load_gather(idx_v, [addr])
                    li_v[lslot, pl.ds(h_i * 128 + bg * 16, 16)] = v

        def gather_start(lslot, sem):
            pltpu.async_copy(table_hbm.at[li_v.at[lslot]],
                             rows_v.at[lslot], sem)

        def gather_wait(lslot, sem):
            pltpu.make_async_copy(table_hbm.at[li_v.at[lslot]],
                                  rows_v.at[lslot], sem).wait()

        def transpose(lslot):
            dst = stg_v.at[lslot]

            @plsc.parallel_loop(0, _CROWS, 1, unroll=8)
            def _(r):
                dh = lane * 0 + (r >> 7)
                d3 = lane * 0 + (r & 127)
                for eg in range(4):
                    v = rows_v[lslot, r, pl.ds(eg * 16, 16)]
                    plsc.store_scatter(dst, [dh, d1[eg], lo8, d3], v)

        def out_start(c, lslot, sem):
            for h_i in range(_HC):
                pltpu.async_copy(stg_v.at[lslot, h_i],
                                 out_hbm.at[2 * c + h_i, :, wid], sem)

        def out_wait(c, lslot, sem):
            for h_i in range(_HC):
                pltpu.make_async_copy(stg_v.at[lslot, h_i],
                                      out_hbm.at[2 * c + h_i, :, wid],
                                      sem).wait()

        build_li(0, 0)
        gather_start(0, gsem0)

        def body(i, carry):
            g = 2 * i
            gather_wait(0, gsem0)
            build_li(g + 1, 1)
            gather_start(1, gsem1)

            @pl.when(i > 0)
            def _():
                out_wait(g - 2, 0, osem0)

            transpose(0)
            out_start(g, 0, osem0)

            gather_wait(1, gsem1)

            @pl.when(i + 1 < _NPAIR)
            def _():
                build_li(g + 2, 0)
                gather_start(0, gsem0)

            @pl.when(i > 0)
            def _():
                out_wait(g - 1, 1, osem1)

            transpose(1)
            out_start(g + 1, 1, osem1)
            return carry

        lax.fori_loop(0, _NPAIR, body, 0)
        out_wait(_NCHUNK - 2, 0, osem0)
        out_wait(_NCHUNK - 1, 1, osem1)

    return gather_kernel


_gather = _make_gather()


def kernel(visit_order, visit_rel_times, pos_table, time_table):
    idx = visit_rel_times.reshape(_B).astype(jnp.int32)
    out5 = _gather(idx, time_table)
    return out5.transpose(2, 4, 0, 1, 3).reshape(_BATCH, _HIST, _EMBED)
